# Initial kernel scaffold; baseline (speedup 1.0000x reference)
import jax
import jax.numpy as jnp
from jax.experimental import pallas as pl


def kernel(table, seq_len):
    raise NotImplementedError("write your pallas kernel here")


# speedup vs baseline: 5.0957x; 5.0957x over previous
import functools

import jax
import jax.numpy as jnp
from jax import lax
from jax.experimental import pallas as pl
from jax.experimental.pallas import tpu as pltpu
from jax.experimental.pallas import tpu_sc as plsc

MAXD = 128
D = 64
S = 1024
G_LEN = 2 * S - 1
MID_LO = S - 1 - MAXD
MID_LEN = 2 * MAXD + 1

_MESH = plsc.VectorSubcoreMesh(core_axis_name="c", subcore_axis_name="s")
_NW = _MESH.num_cores * _MESH.num_subcores
_R = S // _NW


def _body(table_hbm, out_hbm, g_sp, bcast_v, sem):
    c = lax.axis_index("c")
    s = lax.axis_index("s")

    @pl.when(s == 0)
    def _build():
        pltpu.sync_copy(table_hbm, g_sp.at[pl.ds(MID_LO - 1, MID_LEN + 1)])
        pltpu.sync_copy(table_hbm.at[pl.ds(0, 1)], bcast_v.at[pl.ds(0, 1)])
        row0 = [bcast_v[0, pl.ds(16 * k, 16)] for k in range(D // 16)]

        def rep(t, carry):
            for k in range(D // 16):
                bcast_v[t, pl.ds(16 * k, 16)] = row0[k]
            return carry

        lax.fori_loop(1, MID_LO, rep, 0)
        pltpu.sync_copy(bcast_v, g_sp.at[pl.ds(0, MID_LO)])
        pltpu.sync_copy(bcast_v, g_sp.at[pl.ds(MID_LO + MID_LEN, MID_LO)])

    plsc.subcore_barrier()

    wid = s * _MESH.num_cores + c
    base = wid * _R
    copies = []
    for r in range(_R):
        i = base + r
        copies.append(
            pltpu.async_copy(g_sp.at[pl.ds(S - 1 - i, S)], out_hbm.at[i], sem))
    for cp in copies:
        cp.wait()


@jax.jit
def _run(table):
    k = functools.partial(
        pl.kernel,
        out_type=jax.ShapeDtypeStruct((S, S, D), jnp.float32),
        mesh=_MESH,
        compiler_params=pltpu.CompilerParams(use_tc_tiling_on_sc=False),
        scratch_types=[
            pltpu.VMEM_SHARED((G_LEN, D), jnp.float32),
            pltpu.VMEM((MID_LO, D), jnp.float32),
            pltpu.SemaphoreType.DMA,
        ],
    )(_body)
    return k(table.astype(jnp.float32))


def kernel(table, seq_len):
    del seq_len
    return _run(table)


# speedup vs baseline: 24.7681x; 4.8606x over previous
import functools

import jax
import jax.numpy as jnp
from jax import lax
from jax.experimental import pallas as pl
from jax.experimental.pallas import tpu as pltpu
from jax.experimental.pallas import tpu_sc as plsc

MAXD = 128
D = 64
NB = 2 * MAXD + 2
S = 1024
MID_LO = S - 1 - MAXD
MID_LEN = 2 * MAXD + 1
NKT = D // 8
NJT = S // 128
NM = 15

_MESH = plsc.VectorSubcoreMesh(core_axis_name="c", subcore_axis_name="s")
_NW = _MESH.num_cores * _MESH.num_subcores
_PPW = 128 // (_NW // NKT)
_KPT = D // _MESH.num_subcores


def _body(table_hbm, out_hbm, gt_sp, tab_v, row_v, slab_v, i_v, sem0, sem1):
    c = lax.axis_index("c")
    s = lax.axis_index("s")

    pltpu.sync_copy(table_hbm, tab_v)
    k0 = s * _KPT
    lanes = jnp.arange(16, dtype=jnp.int32)
    for kk in range(_KPT):
        col = jnp.full((16,), k0 + kk, dtype=jnp.int32)
        far = plsc.load_gather(tab_v, [jnp.zeros((16,), jnp.int32), col])
        def flank(t, carry):
            row_v[kk, pl.ds(t * 16, 16)] = far
            return carry
        lax.fori_loop(0, 2 * S // 16, flank, 0)
        for m in list(range(0, MID_LEN - 16, 16)) + [MID_LEN - 16]:
            rows = 1 + m + lanes
            row_v[kk, pl.ds(MID_LO + m, 16)] = plsc.load_gather(tab_v, [rows, col])
    pltpu.sync_copy(row_v, gt_sp.at[pl.ds(k0, _KPT)])
    plsc.subcore_barrier()

    wid = s * _MESH.num_cores + c
    kt = wid // (_NW // NKT)
    p0 = (wid % (_NW // NKT)) * _PPW
    pltpu.sync_copy(gt_sp.at[pl.ds(kt * 8, 8)], slab_v)

    sems = (sem0, sem1)

    def drain(b):
        for _ in range(8):
            pltpu.make_async_copy(out_hbm.at[0, 0], i_v.at[b, pl.ds(0, NJT)],
                                  sems[b]).wait()

    def do_phase(p, b):
        buf = i_v.at[b]

        def shift(m, carry):
            for r in range(8):
                for u in range(128 // 16):
                    buf[m, r, pl.ds(16 * u, 16)] = (
                        slab_v[r, pl.ds(p + 128 * m + 16 * u, 16)])
            return carry

        lax.fori_loop(0, NM, shift, 0)
        for q in range(8):
            i = S - 1 - 128 * q - p
            pltpu.async_copy(buf.at[pl.ds(q, NJT)], out_hbm.at[i, kt], sems[b])

    def step(n, carry):
        for b in range(2):
            @pl.when(n > 0)
            def _():
                drain(b)
            do_phase(p0 + 2 * n + b, b)
        return carry

    lax.fori_loop(0, _PPW // 2, step, 0)
    drain(0)
    drain(1)


@jax.jit
def _run(table):
    k = functools.partial(
        pl.kernel,
        out_type=jax.ShapeDtypeStruct((S, NKT, NJT, 8, 128), jnp.float32),
        mesh=_MESH,
        compiler_params=pltpu.CompilerParams(
            use_tc_tiling_on_sc=False, needs_layout_passes=False),
        scratch_types=[
            pltpu.VMEM_SHARED((D, 2 * S), jnp.float32),
            pltpu.VMEM((NB, D), jnp.float32),
            pltpu.VMEM((_KPT, 2 * S), jnp.float32),
            pltpu.VMEM((8, 2 * S), jnp.float32),
            pltpu.VMEM((2, NM, 8, 128), jnp.float32),
            pltpu.SemaphoreType.DMA,
            pltpu.SemaphoreType.DMA,
        ],
    )(_body)
    p5 = k(table.astype(jnp.float32))
    return jnp.transpose(p5, (0, 2, 4, 1, 3)).reshape(S, S, D)


def kernel(table, seq_len):
    del seq_len
    return _run(table)
